# baseline (speedup 1.0000x reference)
import functools
import math

import jax
import jax.numpy as jnp
from jax import lax
from jax.experimental import pallas as pl
from jax.experimental.pallas import tpu as pltpu
from jax.experimental.pallas import tpu_sc as plsc

NC = 2
NS = 16
NW = NC * NS
G = 128
NBUF = 4


def _emb_body(n_tokens, emb, tok_hbm, table_hbm, out_hbm,
              idx_full, rows, gsems, ssems):
    steps = n_tokens // NW // G
    wid = lax.axis_index("s") * NC + lax.axis_index("c")
    idx_row0 = wid * steps
    row0 = wid * steps * G

    pltpu.sync_copy(tok_hbm.at[pl.ds(idx_row0, steps)], idx_full)

    def start_gather(s, b):
        pltpu.async_copy(table_hbm.at[idx_full.at[s]], rows[b], gsems[b])

    def wait_gather(b):
        pltpu.make_async_copy(table_hbm.at[idx_full.at[0]], rows[b],
                              gsems[b]).wait()

    def start_store(s, b):
        pltpu.async_copy(rows[b], out_hbm.at[pl.ds(row0 + s * G, G)], ssems[b])

    def wait_store(b):
        pltpu.make_async_copy(rows[b], out_hbm.at[pl.ds(row0, G)],
                              ssems[b]).wait()

    for b in range(2):
        start_gather(b, b)

    def quad(q, carry):
        for k in range(NBUF):
            s = NBUF * q + k
            b = k
            wait_gather(b)
            start_store(s, b)
            b2 = (k + 2) % NBUF

            @pl.when(s >= 2)
            def _():
                wait_store(b2)

            start_gather(jnp.minimum(s + 2, steps - 1), b2)
        return carry

    lax.fori_loop(0, steps // NBUF, quad, 0)
    for b in ((steps - 2) % NBUF, (steps - 1) % NBUF):
        wait_store(b)
    for b in (steps % NBUF, (steps + 1) % NBUF):
        wait_gather(b)


def kernel(tokens, table):
    bsz, seq = tokens.shape
    _, emb = table.shape
    n_tokens = bsz * seq
    tab128 = jnp.pad(table * math.sqrt(emb), ((0, 0), (0, 128 - emb)))
    tok = tokens.reshape(n_tokens // G, G).astype(jnp.int32)
    mesh = plsc.VectorSubcoreMesh(core_axis_name="c", subcore_axis_name="s",
                                  num_cores=NC, num_subcores=NS)
    steps = n_tokens // NW // G
    run = pl.kernel(
        functools.partial(_emb_body, n_tokens, emb),
        out_type=jax.ShapeDtypeStruct((n_tokens, 128), table.dtype),
        mesh=mesh,
        scratch_types=[
            pltpu.VMEM((steps, G), jnp.int32),
            [pltpu.VMEM((G, 128), jnp.float32) for _ in range(NBUF)],
            [pltpu.SemaphoreType.DMA for _ in range(NBUF)],
            [pltpu.SemaphoreType.DMA for _ in range(NBUF)],
        ],
        compiler_params=pltpu.CompilerParams(use_tc_tiling_on_sc=True),
    )
    out = run(tok, tab128)
    return out[:, :emb].reshape(bsz, seq, emb)


# speedup vs baseline: 1.0726x; 1.0726x over previous
import functools
import math

import jax
import jax.numpy as jnp
from jax import lax
from jax.experimental import pallas as pl
from jax.experimental.pallas import tpu as pltpu
from jax.experimental.pallas import tpu_sc as plsc

NC = 2
NS = 16
NW = NC * NS
LANES = 16
G = 100
NBUF = 4


def _emb_body(n_batches, seq, emb, scale, tok_hbm, table_hbm, out_hbm,
              idx_full, rows, gsems, ssems):
    bpw = n_batches // NW
    wid = lax.axis_index("s") * NC + lax.axis_index("c")
    batch0 = wid * bpw

    pltpu.sync_copy(tok_hbm.at[pl.ds(batch0 * 2, bpw * 2)], idx_full)

    def start_gather(s, b):
        pltpu.async_copy(table_hbm.at[idx_full.at[2 * s]],
                         rows[b].at[0, pl.ds(0, G)], gsems[b])
        pltpu.async_copy(table_hbm.at[idx_full.at[2 * s + 1]],
                         rows[b].at[0, pl.ds(G, G)], gsems[b])

    def wait_gather(b):
        for _ in range(2):
            pltpu.make_async_copy(table_hbm.at[idx_full.at[0]],
                                  rows[b].at[0, pl.ds(0, G)], gsems[b]).wait()

    def scale_rows(b):
        def body(i, carry):
            for r in range(2):
                for j in range(emb // LANES):
                    sl = pl.ds(j * LANES, LANES)
                    rows[b][0, 2 * i + r, sl] = rows[b][0, 2 * i + r, sl] * scale
            return carry

        lax.fori_loop(0, seq // 2, body, 0)

    def start_store(s, b):
        pltpu.async_copy(rows[b], out_hbm.at[pl.ds(batch0 + s, 1)], ssems[b])

    def wait_store(b):
        pltpu.make_async_copy(rows[b], out_hbm.at[pl.ds(batch0, 1)],
                              ssems[b]).wait()

    for b in range(2):
        start_gather(b, b)

    def quad(q, carry):
        for k in range(NBUF):
            s = NBUF * q + k
            b = k
            wait_gather(b)
            scale_rows(b)
            start_store(s, b)
            b2 = (k + 2) % NBUF

            @pl.when(s >= 2)
            def _():
                wait_store(b2)

            start_gather(jnp.minimum(s + 2, bpw - 1), b2)
        return carry

    lax.fori_loop(0, bpw // NBUF, quad, 0)
    for b in ((bpw - 2) % NBUF, (bpw - 1) % NBUF):
        wait_store(b)
    for b in (bpw % NBUF, (bpw + 1) % NBUF):
        wait_gather(b)


def kernel(tokens, table):
    bsz, seq = tokens.shape
    _, emb = table.shape
    tok = tokens.reshape(bsz * seq // G, G).astype(jnp.int32)
    mesh = plsc.VectorSubcoreMesh(core_axis_name="c", subcore_axis_name="s",
                                  num_cores=NC, num_subcores=NS)
    run = pl.kernel(
        functools.partial(_emb_body, bsz, seq, emb, math.sqrt(emb)),
        out_type=jax.ShapeDtypeStruct((bsz, seq, emb), table.dtype),
        mesh=mesh,
        scratch_types=[
            pltpu.VMEM((bsz * seq // G // NW, G), jnp.int32),
            [pltpu.VMEM((1, seq, emb), jnp.float32) for _ in range(NBUF)],
            [pltpu.SemaphoreType.DMA for _ in range(NBUF)],
            [pltpu.SemaphoreType.DMA for _ in range(NBUF)],
        ],
        compiler_params=pltpu.CompilerParams(use_tc_tiling_on_sc=False),
    )
    return run(tok, table)
